# baseline (speedup 1.0000x reference)
import dataclasses
import functools

import jax
import jax.numpy as jnp
from jax import lax
from jax.experimental import pallas as pl
from jax.experimental.pallas import tpu as pltpu
from jax.experimental.pallas import tpu_sc as plsc

L = 16
G = 128
NG = 8
W = G * NG


@functools.lru_cache(maxsize=None)
def _make_gather(Vt, D, N, Vf):
    mesh = plsc.VectorSubcoreMesh(core_axis_name="c", subcore_axis_name="s")
    cp = pltpu.CompilerParams(use_tc_tiling_on_sc=False)
    if "needs_layout_passes" in pltpu.CompilerParams.__dataclass_fields__:
        cp = dataclasses.replace(cp, needs_layout_passes=False)

    @functools.partial(
        pl.kernel,
        out_type=jax.ShapeDtypeStruct((N, D), jnp.float32),
        mesh=mesh,
        compiler_params=cp,
        scratch_types=[
            pltpu.VMEM((Vf, D), jnp.float32),
            pltpu.VMEM((W,), jnp.int32),
            pltpu.SemaphoreType.DMA,
        ],
    )
    def gather_kernel(table_hbm, frozen_hbm, idx_hbm, out_hbm, frozen_v,
                      cidx_v, gsem):
        pltpu.sync_copy(frozen_hbm, frozen_v)

        def body(i_vmem, o_vmem):
            copies = []
            mx = jnp.zeros((L,), jnp.int32)
            for g in range(NG):
                def clamp_step(k, mx, g=g):
                    v = i_vmem[0, pl.ds(g * G + k * L, L)]
                    cidx_v[pl.ds(g * G + k * L, L)] = jnp.minimum(v, Vt - 1)
                    return jnp.maximum(mx, v)

                mx = lax.fori_loop(0, G // L, clamp_step, mx)
                copies.append(pltpu.async_copy(
                    table_hbm.at[cidx_v.at[pl.ds(g * G, G)]],
                    o_vmem.at[pl.ds(g * G, G)],
                    gsem,
                ))
            for c in copies:
                c.wait()
            any_frozen = jnp.max(mx) >= Vt

            @pl.when(any_frozen)
            def _fixup():
                def group(k, _):
                    v = i_vmem[0, pl.ds(k * L, L)]
                    msk = v >= Vt
                    fr = jnp.clip(v - Vt, 0, Vf - 1)
                    rows = lax.iota(jnp.int32, L) + k * L

                    def col(c, _):
                        cvec = jnp.zeros((L,), jnp.int32) + c
                        vals = plsc.load_gather(frozen_v, [fr, cvec], mask=msk)
                        plsc.store_scatter(o_vmem, [rows, cvec], vals, mask=msk)
                        return 0

                    return lax.fori_loop(0, D, col, 0)

                lax.fori_loop(0, W // L, group, 0)

        pltpu.emit_pipeline(
            body,
            grid=(N // W,),
            in_specs=[pl.BlockSpec((1, W), lambda i: (0, i))],
            out_specs=[pl.BlockSpec((W, D), lambda i: (i, 0))],
            core_axis_name=("c", "s"),
            dimension_semantics=(pltpu.PARALLEL,),
        )(idx_hbm, out_hbm)

    return gather_kernel


def _transpose_table(table):
    V, D = table.shape
    B = 512
    nblk = (V + B - 1) // B
    table_t = table.T

    def body(x_ref, o_ref):
        xT = x_ref[...].T
        pp = lax.broadcasted_iota(jnp.int32, (128, B), 0)
        ll = lax.broadcasted_iota(jnp.int32, (128, B), 1)
        for j in range(4):
            sel = (ll == 4 * pp + j).astype(jnp.float32)
            o_ref[0, :, j * D:(j + 1) * D] = jnp.dot(
                sel, xT, preferred_element_type=jnp.float32)

    out3 = pl.pallas_call(
        body,
        grid=(nblk,),
        in_specs=[pl.BlockSpec((D, B), lambda i: (0, i))],
        out_specs=pl.BlockSpec((1, 128, 128), lambda i: (i, 0, 0)),
        out_shape=jax.ShapeDtypeStruct((nblk, 128, 128), jnp.float32),
    )(table_t)
    return out3.reshape(nblk * 128 * 4, D)


@jax.jit
def kernel(x, trainable_embedding, frozen_embedding):
    B, H = x.shape
    Vt, D = trainable_embedding.shape
    Vf = frozen_embedding.shape[0]
    N = B * H
    idx = x.reshape(1, N).astype(jnp.int32)
    table_lin = _transpose_table(trainable_embedding)
    out = _make_gather(Vt, D, N, Vf)(table_lin, frozen_embedding, idx)
    return out.reshape(B, H, D)


# speedup vs baseline: 1.3749x; 1.3749x over previous
import dataclasses
import functools

import jax
import jax.numpy as jnp
from jax import lax
from jax.experimental import pallas as pl
from jax.experimental.pallas import tpu as pltpu
from jax.experimental.pallas import tpu_sc as plsc

L = 16
W = 128


@functools.lru_cache(maxsize=None)
def _make_gather(Vt, D, N, Vf):
    mesh = plsc.VectorSubcoreMesh(core_axis_name="c", subcore_axis_name="s")
    cp = pltpu.CompilerParams(use_tc_tiling_on_sc=False)
    if "needs_layout_passes" in pltpu.CompilerParams.__dataclass_fields__:
        cp = dataclasses.replace(cp, needs_layout_passes=False)

    @functools.partial(
        pl.kernel,
        out_type=jax.ShapeDtypeStruct((N, D), jnp.float32),
        mesh=mesh,
        compiler_params=cp,
        scratch_types=[
            pltpu.VMEM((Vf, D), jnp.float32),
            pltpu.VMEM((W,), jnp.int32),
        ],
    )
    def gather_kernel(table_hbm, frozen_hbm, idx_hbm, out_hbm, frozen_v, cidx_v):
        pltpu.sync_copy(frozen_hbm, frozen_v)

        def body(i_vmem, o_vmem):
            def clamp_step(k, mx):
                v = i_vmem[0, pl.ds(k * L, L)]
                cidx_v[pl.ds(k * L, L)] = jnp.minimum(v, Vt - 1)
                return jnp.maximum(mx, v)

            mx = lax.fori_loop(0, W // L, clamp_step, jnp.zeros((L,), jnp.int32))
            pltpu.sync_copy(table_hbm.at[cidx_v], o_vmem)
            any_frozen = jnp.max(mx) >= Vt

            @pl.when(any_frozen)
            def _fixup():
                def group(k, _):
                    v = i_vmem[0, pl.ds(k * L, L)]
                    msk = v >= Vt
                    fr = jnp.clip(v - Vt, 0, Vf - 1)
                    rows = lax.iota(jnp.int32, L) + k * L

                    def col(c, _):
                        cvec = jnp.zeros((L,), jnp.int32) + c
                        vals = plsc.load_gather(frozen_v, [fr, cvec], mask=msk)
                        plsc.store_scatter(o_vmem, [rows, cvec], vals, mask=msk)
                        return 0

                    return lax.fori_loop(0, D, col, 0)

                lax.fori_loop(0, W // L, group, 0)

        pltpu.emit_pipeline(
            body,
            grid=(N // W,),
            in_specs=[pl.BlockSpec((1, W), lambda i: (0, i))],
            out_specs=[pl.BlockSpec((W, D), lambda i: (i, 0))],
            core_axis_name=("c", "s"),
            dimension_semantics=(pltpu.PARALLEL,),
        )(idx_hbm, out_hbm)

    return gather_kernel


@jax.jit
def kernel(x, trainable_embedding, frozen_embedding):
    B, H = x.shape
    Vt, D = trainable_embedding.shape
    Vf = frozen_embedding.shape[0]
    N = B * H
    idx = x.reshape(1, N).astype(jnp.int32)
    out = _make_gather(Vt, D, N, Vf)(trainable_embedding, frozen_embedding, idx)
    return out.reshape(B, H, D)
